# baseline (speedup 1.0000x reference)
import functools

import jax
import jax.numpy as jnp
from jax import lax
from jax.experimental import pallas as pl
from jax.experimental.pallas import tpu as pltpu
from jax.experimental.pallas import tpu_sc as plsc

_MAX_RANGE = 1000
_BATCH = 16384
_COLS = 128
_NUM_CORES = 2
_NUM_SUBCORES = 16
_NW = _NUM_CORES * _NUM_SUBCORES
_ROWS_PER_W = (_BATCH // _COLS) // _NW
_MAGIC = 8388608.0


def _body(x_hbm, param_hbm, out_hbm, x_v, idx_v, gat_v, sem):
    wid = lax.axis_index("s") * _NUM_CORES + lax.axis_index("c")
    row0 = wid * _ROWS_PER_W
    pltpu.sync_copy(x_hbm.at[pl.ds(row0, _ROWS_PER_W)], x_v)
    lane = lax.iota(jnp.int32, 16)
    for j in range(_ROWS_PER_W):
        for c in range(_COLS // 16):
            xv = x_v[j, pl.ds(c * 16, 16)]
            xs = xv * float(_MAX_RANGE - 1)
            rounded = (xs + _MAGIC) - _MAGIC
            col = rounded.astype(jnp.int32)
            base = (row0 + j) * _COLS + c * 16
            idx_v[j, pl.ds(c * 16, 16)] = (base + lane) * _MAX_RANGE + col
    copies = [
        pltpu.async_copy(param_hbm.at[idx_v.at[j]], gat_v.at[j], sem)
        for j in range(_ROWS_PER_W)
    ]
    for cp in copies:
        cp.wait()
    pltpu.sync_copy(gat_v, out_hbm.at[pl.ds(row0, _ROWS_PER_W)])


@functools.partial(
    pl.kernel,
    mesh=plsc.VectorSubcoreMesh(core_axis_name="c", subcore_axis_name="s"),
    out_type=jax.ShapeDtypeStruct((_BATCH // _COLS, _COLS), jnp.float32),
    scratch_types=[
        pltpu.VMEM((_ROWS_PER_W, _COLS), jnp.float32),
        pltpu.VMEM((_ROWS_PER_W, _COLS), jnp.int32),
        pltpu.VMEM((_ROWS_PER_W, _COLS), jnp.float32),
        pltpu.SemaphoreType.DMA,
    ],
)
def _gather(x_hbm, param_hbm, out_hbm, x_v, idx_v, gat_v, sem):
    _body(x_hbm, param_hbm, out_hbm, x_v, idx_v, gat_v, sem)


def kernel(x, param):
    x2 = x.reshape(_BATCH // _COLS, _COLS)
    pflat = param.reshape(_BATCH * _MAX_RANGE)
    out = _gather(x2, pflat)
    return out.reshape(_BATCH, 1, 1)
